# Initial kernel scaffold; baseline (speedup 1.0000x reference)
import jax
import jax.numpy as jnp
from jax.experimental import pallas as pl


def kernel(x_low, x_high, attention, WSI):
    raise NotImplementedError("write your pallas kernel here")


# speedup vs baseline: 6.2188x; 6.2188x over previous
import functools

import jax
import jax.numpy as jnp
from jax import lax
from jax.experimental import pallas as pl
from jax.experimental.pallas import tpu as pltpu
from jax.experimental.pallas import tpu_sc as plsc

N_PATCHES = 200
AH = AW = 128
H = W = 2048
C = 3
PATCH = 32
SY = H // AH
CHUNK = 16
WPC = W // CHUNK
PATLEN = C * PATCH * 2
ROWS = 2 * N_PATCHES * PATLEN
NC, NS = 2, 16
NW = NC * NS
RPW = ROWS // NW
GCH = 120
NGC = RPW // GCH
KPAD = 256


def _topk_body(score_ref, p_ref, sa_ref, ridx_ref):
    b = pl.program_id(0)
    pm = p_ref[0]
    pos = (lax.broadcasted_iota(jnp.int32, (AH, AW), 0) * AW
           + lax.broadcasted_iota(jnp.int32, (AH, AW), 1))
    lane = lax.broadcasted_iota(jnp.int32, (KPAD,), 0)

    def body(j, st):
        s, idxv, sav = st
        m = jnp.max(s)
        chosen = jnp.min(jnp.where(s == m, pos, jnp.int32(1 << 30)))
        hit = pos == chosen
        pv = jnp.sum(jnp.where(hit, pm, 0.0))
        s = jnp.where(hit, jnp.float32(-1e30), s)
        idxv = jnp.where(lane == j, chosen, idxv)
        sav = jnp.where(lane == j, pv, sav)
        return s, idxv, sav

    _, idxv, sav = lax.fori_loop(
        0, N_PATCHES, body,
        (score_ref[0], jnp.zeros((KPAD,), jnp.int32),
         jnp.zeros((KPAD,), jnp.float32)))

    ys = idxv // AW
    xs = idxv - ys * AW
    y0 = jnp.minimum(ys * SY, H - PATCH)
    x0c = jnp.minimum(xs, WPC - 2)
    off = b * (C * H * WPC) + y0 * WPC + x0c
    w = lax.broadcasted_iota(jnp.int32, (PATLEN,), 0)
    c = w // (PATCH * 2)
    rem = w - c * (PATCH * 2)
    r = rem // 2
    k = rem - r * 2
    pat = c * (H * WPC) + r * WPC + k
    ridx_ref[0] = off[:, None] + pat[None, :]
    sa_ref[0, 0] = sav


def _topk_call(score, p):
    return pl.pallas_call(
        _topk_body,
        grid=(2,),
        in_specs=[pl.BlockSpec((1, AH, AW), lambda b: (b, 0, 0)),
                  pl.BlockSpec((1, AH, AW), lambda b: (b, 0, 0))],
        out_specs=[pl.BlockSpec((1, 1, KPAD), lambda b: (b, 0, 0)),
                   pl.BlockSpec((1, KPAD, PATLEN), lambda b: (b, 0, 0))],
        out_shape=[jax.ShapeDtypeStruct((2, 1, KPAD), jnp.float32),
                   jax.ShapeDtypeStruct((2, KPAD, PATLEN), jnp.int32)],
    )(score, p)


@functools.cache
def _make_gather():
    mesh = plsc.VectorSubcoreMesh(core_axis_name="c", subcore_axis_name="s")

    @functools.partial(
        pl.kernel,
        mesh=mesh,
        out_type=jax.ShapeDtypeStruct((ROWS, CHUNK), jnp.float32),
        compiler_params=pltpu.CompilerParams(use_tc_tiling_on_sc=False),
        scratch_types=[
            pltpu.VMEM((RPW,), jnp.int32),
            pltpu.VMEM((RPW, CHUNK), jnp.float32),
            pltpu.SemaphoreType.DMA,
        ],
    )
    def gather_k(table_hbm, ridx_hbm, out_hbm, idx_v, rows_v, sem):
        wid = lax.axis_index("s") * NC + lax.axis_index("c")
        base = wid * RPW
        pltpu.sync_copy(ridx_hbm.at[pl.ds(base, RPW)], idx_v)
        cps = [
            pltpu.async_copy(table_hbm.at[idx_v.at[pl.ds(g * GCH, GCH)]],
                             rows_v.at[pl.ds(g * GCH, GCH)], sem)
            for g in range(NGC)
        ]
        for cp in cps:
            cp.wait()
        pltpu.sync_copy(rows_v, out_hbm.at[pl.ds(base, RPW)])

    return gather_k


def kernel(x_low, x_high, attention, WSI):
    B = attention.shape[0]
    flat = attention.reshape(B, -1)
    p = flat / jnp.sum(flat, axis=-1, keepdims=True)
    logp = jnp.log(p + 1e-12)
    u = jax.random.uniform(jax.random.key(42), flat.shape,
                           minval=1e-9, maxval=1.0)
    gumbel = -jnp.log(-jnp.log(u))
    score = logp + gumbel
    sa_pad, ridx_pad = _topk_call(score.reshape(B, AH, AW),
                                  p.reshape(B, AH, AW))
    ridx = ridx_pad[:, :N_PATCHES, :].reshape(-1)
    table = WSI.reshape(B * C * H * WPC, CHUNK)
    rows = _make_gather()(table, ridx)
    patches = rows.reshape(B, N_PATCHES, C, PATCH, PATCH)
    return patches, sa_pad[:, 0, :N_PATCHES]
